# baseline (speedup 1.0000x reference)
import functools

import jax
import jax.numpy as jnp
from jax import lax
from jax.experimental import pallas as pl
from jax.experimental.pallas import tpu as pltpu
from jax.experimental.pallas import tpu_sc as plsc

NUM_CLASSES = 1000000
D_MODEL = 64
BATCH = 4096
SEQ = 200

_NC = 2
_NS = 16
_NW = _NC * _NS

_B = BATCH * SEQ
_CHUNK = 128
_PER_W = _B // _NW
_NCHUNK = _PER_W // _CHUNK
_NBUF = 4


def _body(idx_hbm, table_hbm, out_hbm, idx_v, bufs, gsems, osems):
  wid = lax.axis_index("s") * _NC + lax.axis_index("c")
  base = wid * _PER_W

  pltpu.sync_copy(idx_hbm.at[wid], idx_v)

  for b in range(_NBUF):
    pltpu.async_copy(table_hbm.at[idx_v.at[b]], bufs.at[b], gsems.at[b])

  @pl.loop(0, _NCHUNK, step=_NBUF)
  def _(c0):
    for b in range(_NBUF):
      c = c0 + b
      pltpu.make_async_copy(table_hbm.at[idx_v.at[b]], bufs.at[b],
                            gsems.at[b]).wait()
      pltpu.async_copy(bufs.at[b], out_hbm.at[pl.ds(base + c * _CHUNK, _CHUNK)],
                       osems.at[b])

    for b in range(_NBUF):
      c = c0 + b
      nxt = c + _NBUF
      pltpu.make_async_copy(bufs.at[b],
                            out_hbm.at[pl.ds(base + c * _CHUNK, _CHUNK)],
                            osems.at[b]).wait()

      @pl.when(nxt < _NCHUNK)
      def _():
        pltpu.async_copy(table_hbm.at[idx_v.at[nxt]], bufs.at[b],
                         gsems.at[b])


@functools.partial(jax.jit, static_argnames=())
def _run(classes_flat, class_embedding):
  idx3 = classes_flat.reshape(_NW, _NCHUNK, _CHUNK)
  f = pl.kernel(
      _body,
      out_type=jax.ShapeDtypeStruct((_B, D_MODEL), jnp.float32),
      mesh=plsc.VectorSubcoreMesh(core_axis_name="c", subcore_axis_name="s"),
      compiler_params=pltpu.CompilerParams(use_tc_tiling_on_sc=False),
      scratch_types=[
          pltpu.VMEM((_NCHUNK, _CHUNK), jnp.int32),
          pltpu.VMEM((_NBUF, _CHUNK, D_MODEL), jnp.float32),
          pltpu.SemaphoreType.DMA((_NBUF,)),
          pltpu.SemaphoreType.DMA((_NBUF,)),
      ],
  )
  return f(idx3, class_embedding)


def kernel(classes, bbs, class_embedding):
  del bbs
  out = _run(classes.reshape(-1).astype(jnp.int32), class_embedding)
  return out.reshape(BATCH, SEQ, D_MODEL)


# speedup vs baseline: 1.0052x; 1.0052x over previous
import functools

import jax
import jax.numpy as jnp
from jax import lax
from jax.experimental import pallas as pl
from jax.experimental.pallas import tpu as pltpu
from jax.experimental.pallas import tpu_sc as plsc

NUM_CLASSES = 1000000
D_MODEL = 64
BATCH = 4096
SEQ = 200

_NC = 2
_NS = 16
_NW = _NC * _NS

_B = BATCH * SEQ
_CHUNK = 128
_PER_W = _B // _NW
_NCHUNK = _PER_W // _CHUNK
_NBUF = 8


def _body(idx_hbm, table_hbm, out_hbm, idx_v, bufs, gsems, osems):
  wid = lax.axis_index("s") * _NC + lax.axis_index("c")
  base = wid * _PER_W

  pltpu.sync_copy(idx_hbm.at[wid], idx_v)

  def start_gather(c, b):
    pltpu.async_copy(table_hbm.at[idx_v.at[c]], bufs.at[b], gsems.at[b])

  def wait_gather(c, b):
    pltpu.make_async_copy(table_hbm.at[idx_v.at[c]], bufs.at[b],
                          gsems.at[b]).wait()

  def start_write(c, b):
    pltpu.async_copy(bufs.at[b], out_hbm.at[pl.ds(base + c * _CHUNK, _CHUNK)],
                     osems.at[b])

  def wait_write(c, b):
    pltpu.make_async_copy(bufs.at[b],
                          out_hbm.at[pl.ds(base + c * _CHUNK, _CHUNK)],
                          osems.at[b]).wait()

  for b in range(_NBUF):
    start_gather(b, b)

  @pl.loop(0, _NCHUNK, step=_NBUF)
  def _(c0):
    for b in range(_NBUF):
      c = c0 + b
      wait_gather(c, b)
      start_write(c, b)
      pb = (b - 1) % _NBUF
      pc = c - 1
      nxt = pc + _NBUF

      @pl.when(jnp.logical_and(pc >= 0, nxt < _NCHUNK))
      def _():
        wait_write(pc, pb)
        start_gather(nxt, pb)

  for b in range(_NBUF):
    c = _NCHUNK - _NBUF + b
    wait_write(c, c % _NBUF)


@functools.partial(jax.jit, static_argnames=())
def _run(classes_flat, class_embedding):
  idx3 = classes_flat.reshape(_NW, _NCHUNK, _CHUNK)
  f = pl.kernel(
      _body,
      out_type=jax.ShapeDtypeStruct((_B, D_MODEL), jnp.float32),
      mesh=plsc.VectorSubcoreMesh(core_axis_name="c", subcore_axis_name="s"),
      compiler_params=pltpu.CompilerParams(use_tc_tiling_on_sc=False),
      scratch_types=[
          pltpu.VMEM((_NCHUNK, _CHUNK), jnp.int32),
          pltpu.VMEM((_NBUF, _CHUNK, D_MODEL), jnp.float32),
          pltpu.SemaphoreType.DMA((_NBUF,)),
          pltpu.SemaphoreType.DMA((_NBUF,)),
      ],
  )
  return f(idx3, class_embedding)


def kernel(classes, bbs, class_embedding):
  del bbs
  out = _run(classes.reshape(-1).astype(jnp.int32), class_embedding)
  return out.reshape(BATCH, SEQ, D_MODEL)
